# baseline (speedup 1.0000x reference)
import functools

import jax
import jax.numpy as jnp
from jax import lax
from jax.experimental import pallas as pl

N = 4096
INPUT_DIM = 768
CODE_DIM = 256
K = 8192
BLK = 256


def _nn_from_d(d):
    dmin = jnp.min(d, axis=1, keepdims=True)
    iota = lax.broadcasted_iota(jnp.int32, d.shape, 1)
    return jnp.min(jnp.where(d == dmin, iota, K), axis=1)


def _distance(ze, emb):
    znorm = jnp.sum(ze * ze, axis=1, keepdims=True)
    ones = jnp.ones((1, CODE_DIM), jnp.float32)
    enorm = lax.dot_general(ones, emb * emb, (((1,), (1,)), ((), ())),
                            preferred_element_type=jnp.float32)
    cross = lax.dot_general(ze, emb, (((1,), (1,)), ((), ())),
                            preferred_element_type=jnp.float32)
    return (znorm + enorm) - 2.0 * cross


def _gather_rows(nn, emb):
    iota = lax.broadcasted_iota(jnp.int32, (nn.shape[0], K), 1)
    oh = (iota == nn[:, None]).astype(jnp.float32)
    return lax.dot_general(oh, emb, (((1,), (0,)), ((), ())),
                           preferred_element_type=jnp.float32,
                           precision=lax.Precision.HIGHEST)


def _stage1_body(x_ref, w_ref, b_ref, emb_ref,
                 ze1_ref, nn1_ref, zq1_ref, ze2_ref):
    ze1 = jnp.dot(x_ref[...], w_ref[...],
                  preferred_element_type=jnp.float32) + b_ref[...]
    ze1_ref[...] = ze1
    emb = emb_ref[...]
    nn = _nn_from_d(_distance(ze1, emb))
    nn1_ref[...] = nn[:, None]
    zq = _gather_rows(nn, emb)
    zq1_ref[...] = zq
    ze2_ref[...] = ze1 - zq


def _stage_body(ze_ref, emb_ref, nn_ref, zq_ref, zenext_ref):
    ze = ze_ref[...]
    emb = emb_ref[...]
    nn = _nn_from_d(_distance(ze, emb))
    nn_ref[...] = nn[:, None]
    zq = _gather_rows(nn, emb)
    zq_ref[...] = zq
    zenext_ref[...] = ze - zq


def _stage4_body(ze_ref, emb_ref, nn_ref, zq_ref):
    ze = ze_ref[...]
    emb = emb_ref[...]
    nn = _nn_from_d(_distance(ze, emb))
    nn_ref[...] = nn[:, None]
    zq_ref[...] = _gather_rows(nn, emb)


def _dec_body(zq1_ref, zq2_ref, zq3_ref, zq4_ref, w_ref, b_ref, out_ref):
    s = (zq1_ref[...] + zq2_ref[...]) + (zq3_ref[...] + zq4_ref[...])
    out_ref[...] = jnp.dot(s, w_ref[...],
                           preferred_element_type=jnp.float32) + b_ref[...]


def _row_spec(cols):
    return pl.BlockSpec((BLK, cols), lambda i: (i, 0))


def _full_spec(rows, cols):
    return pl.BlockSpec((rows, cols), lambda i: (0, 0))


_GRID = N // BLK
_ZE = jax.ShapeDtypeStruct((N, CODE_DIM), jnp.float32)
_NN = jax.ShapeDtypeStruct((N, 1), jnp.int32)


@jax.jit
def kernel(x, enc_w, enc_b, emb1, emb2, emb3, emb4, emb5, dec_w, dec_b):
    del emb5
    enc_b2 = enc_b.reshape(1, CODE_DIM)
    dec_b2 = dec_b.reshape(1, INPUT_DIM)

    ze1, nn1, zq1, ze2 = pl.pallas_call(
        _stage1_body,
        grid=(_GRID,),
        in_specs=[_row_spec(INPUT_DIM), _full_spec(INPUT_DIM, CODE_DIM),
                  _full_spec(1, CODE_DIM), _full_spec(K, CODE_DIM)],
        out_specs=[_row_spec(CODE_DIM), _row_spec(1), _row_spec(CODE_DIM),
                   _row_spec(CODE_DIM)],
        out_shape=[_ZE, _NN, _ZE, _ZE],
    )(x, enc_w, enc_b2, emb1)

    stage = pl.pallas_call(
        _stage_body,
        grid=(_GRID,),
        in_specs=[_row_spec(CODE_DIM), _full_spec(K, CODE_DIM)],
        out_specs=[_row_spec(1), _row_spec(CODE_DIM), _row_spec(CODE_DIM)],
        out_shape=[_NN, _ZE, _ZE],
    )
    nn2, zq2, ze3 = stage(ze2, emb2)
    nn3, zq3, ze4 = stage(ze3, emb3)

    nn4, zq4 = pl.pallas_call(
        _stage4_body,
        grid=(_GRID,),
        in_specs=[_row_spec(CODE_DIM), _full_spec(K, CODE_DIM)],
        out_specs=[_row_spec(1), _row_spec(CODE_DIM)],
        out_shape=[_NN, _ZE],
    )(ze4, emb4)

    x_hat = pl.pallas_call(
        _dec_body,
        grid=(_GRID,),
        in_specs=[_row_spec(CODE_DIM)] * 4
        + [_full_spec(CODE_DIM, INPUT_DIM), _full_spec(1, INPUT_DIM)],
        out_specs=_row_spec(INPUT_DIM),
        out_shape=jax.ShapeDtypeStruct((N, INPUT_DIM), jnp.float32),
    )(zq1, zq2, zq3, zq4, dec_w, dec_b2)

    r = lambda a: a.reshape(N)
    return (x_hat, ze1, ze2, ze3, ze4, zq1, zq2, zq3, zq4,
            r(nn1), r(nn2), r(nn3), r(nn4))


# speedup vs baseline: 1.4851x; 1.4851x over previous
import functools

import jax
import jax.numpy as jnp
from jax import lax
from jax.experimental import pallas as pl
from jax.experimental.pallas import tpu as pltpu
from jax.experimental.pallas import tpu_sc as plsc

N = 4096
INPUT_DIM = 768
CODE_DIM = 256
K = 8192
BLK = 256

_SC_WORKERS = 32
_B_PER_W = N // _SC_WORKERS


def _nn_from_d(d):
    dmin = jnp.min(d, axis=1, keepdims=True)
    iota = lax.broadcasted_iota(jnp.int32, d.shape, 1)
    return jnp.min(jnp.where(d == dmin, iota, K), axis=1)


def _distance(ze, emb):
    znorm = jnp.sum(ze * ze, axis=1, keepdims=True)
    ones = jnp.ones((1, CODE_DIM), jnp.float32)
    enorm = lax.dot_general(ones, emb * emb, (((1,), (1,)), ((), ())),
                            preferred_element_type=jnp.float32,
                            precision=lax.Precision.HIGHEST)
    cross = lax.dot_general(ze, emb, (((1,), (1,)), ((), ())),
                            preferred_element_type=jnp.float32)
    return (znorm + enorm) - 2.0 * cross


def _stage1_body(x_ref, w_ref, b_ref, emb_ref, ze1_ref, nn1_ref):
    ze1 = jnp.dot(x_ref[...], w_ref[...],
                  preferred_element_type=jnp.float32) + b_ref[...]
    ze1_ref[...] = ze1
    nn = _nn_from_d(_distance(ze1, emb_ref[...]))
    nn1_ref[...] = nn[:, None]


def _stage_body(zeprev_ref, zqprev_ref, emb_ref, ze_ref, nn_ref):
    ze = zeprev_ref[...] - zqprev_ref[...]
    ze_ref[...] = ze
    nn = _nn_from_d(_distance(ze, emb_ref[...]))
    nn_ref[...] = nn[:, None]


@functools.cache
def _make_sc_gather():
    mesh = plsc.VectorSubcoreMesh(core_axis_name="c", subcore_axis_name="s")

    @functools.partial(
        pl.kernel,
        out_type=jax.ShapeDtypeStruct((N, CODE_DIM), jnp.float32),
        mesh=mesh,
        scratch_types=[
            pltpu.VMEM((_B_PER_W,), jnp.int32),
            pltpu.VMEM((_B_PER_W, CODE_DIM), jnp.float32),
            pltpu.SemaphoreType.DMA,
        ],
    )
    def _sc_gather(table_hbm, idx_hbm, out_hbm, idx_v, rows_v, sem):
        wid = lax.axis_index("s") * 2 + lax.axis_index("c")
        base = wid * _B_PER_W
        pltpu.sync_copy(idx_hbm.at[pl.ds(base, _B_PER_W)], idx_v)
        pltpu.async_copy(table_hbm.at[idx_v], rows_v, sem).wait()
        pltpu.sync_copy(rows_v, out_hbm.at[pl.ds(base, _B_PER_W)])

    return _sc_gather


def _dec_body(ze1_ref, zq1_ref, zq2_ref, zq3_ref, zq4_ref,
              w_ref, b_ref, out_ref):
    ze1 = ze1_ref[...]
    s = ((zq1_ref[...] + zq2_ref[...]) + zq3_ref[...]) + zq4_ref[...]
    di = ze1 + (-ze1 + s)
    out_ref[...] = jnp.dot(di, w_ref[...],
                           preferred_element_type=jnp.float32) + b_ref[...]


def _row_spec(cols):
    return pl.BlockSpec((BLK, cols), lambda i: (i, 0))


def _full_spec(rows, cols):
    return pl.BlockSpec((rows, cols), lambda i: (0, 0))


_GRID = N // BLK
_ZE = jax.ShapeDtypeStruct((N, CODE_DIM), jnp.float32)
_NN = jax.ShapeDtypeStruct((N, 1), jnp.int32)


@jax.jit
def kernel(x, enc_w, enc_b, emb1, emb2, emb3, emb4, emb5, dec_w, dec_b):
    del emb5
    enc_b2 = enc_b.reshape(1, CODE_DIM)
    dec_b2 = dec_b.reshape(1, INPUT_DIM)

    ze1, nn1 = pl.pallas_call(
        _stage1_body,
        grid=(_GRID,),
        in_specs=[_row_spec(INPUT_DIM), _full_spec(INPUT_DIM, CODE_DIM),
                  _full_spec(1, CODE_DIM), _full_spec(K, CODE_DIM)],
        out_specs=[_row_spec(CODE_DIM), _row_spec(1)],
        out_shape=[_ZE, _NN],
    )(x, enc_w, enc_b2, emb1)
    zq1 = _make_sc_gather()(emb1, nn1.reshape(N))

    stage = pl.pallas_call(
        _stage_body,
        grid=(_GRID,),
        in_specs=[_row_spec(CODE_DIM), _row_spec(CODE_DIM),
                  _full_spec(K, CODE_DIM)],
        out_specs=[_row_spec(CODE_DIM), _row_spec(1)],
        out_shape=[_ZE, _NN],
    )
    ze2, nn2 = stage(ze1, zq1, emb2)
    zq2 = _make_sc_gather()(emb2, nn2.reshape(N))
    ze3, nn3 = stage(ze2, zq2, emb3)
    zq3 = _make_sc_gather()(emb3, nn3.reshape(N))
    ze4, nn4 = stage(ze3, zq3, emb4)
    zq4 = _make_sc_gather()(emb4, nn4.reshape(N))

    x_hat = pl.pallas_call(
        _dec_body,
        grid=(_GRID,),
        in_specs=[_row_spec(CODE_DIM)] * 5
        + [_full_spec(CODE_DIM, INPUT_DIM), _full_spec(1, INPUT_DIM)],
        out_specs=_row_spec(INPUT_DIM),
        out_shape=jax.ShapeDtypeStruct((N, INPUT_DIM), jnp.float32),
    )(ze1, zq1, zq2, zq3, zq4, dec_w, dec_b2)

    r = lambda a: a.reshape(N)
    return (x_hat, ze1, ze2, ze3, ze4, zq1, zq2, zq3, zq4,
            r(nn1), r(nn2), r(nn3), r(nn4))


# speedup vs baseline: 3.2614x; 2.1961x over previous
import functools

import jax
import jax.numpy as jnp
from jax import lax
from jax.experimental import pallas as pl
from jax.experimental.pallas import tpu as pltpu
from jax.experimental.pallas import tpu_sc as plsc

N = 4096
INPUT_DIM = 768
CODE_DIM = 256
K = 8192
BLK = 256

_SC_WORKERS = 32
_B_PER_W = N // _SC_WORKERS


def _nn_from_d(d):
    dmin = jnp.min(d, axis=1, keepdims=True)
    iota = lax.broadcasted_iota(jnp.int32, d.shape, 1)
    return jnp.min(jnp.where(d == dmin, iota, K), axis=1)


def _distance(ze, emb, enorm):
    znorm = jnp.sum(ze * ze, axis=1, keepdims=True)
    cross = lax.dot_general(ze, emb, (((1,), (1,)), ((), ())),
                            preferred_element_type=jnp.float32)
    return (znorm + enorm) - 2.0 * cross


def _enorm_body(e1_ref, e2_ref, e3_ref, e4_ref, o1_ref, o2_ref, o3_ref, o4_ref):
    ones = jnp.ones((1, CODE_DIM), jnp.float32)
    for e_ref, o_ref in ((e1_ref, o1_ref), (e2_ref, o2_ref),
                         (e3_ref, o3_ref), (e4_ref, o4_ref)):
        emb = e_ref[...]
        o_ref[...] = lax.dot_general(ones, emb * emb, (((1,), (1,)), ((), ())),
                                     preferred_element_type=jnp.float32,
                                     precision=lax.Precision.HIGHEST)


def _stage1_body(x_ref, w_ref, b_ref, emb_ref, en_ref, ze1_ref, nn1_ref):
    ze1 = jnp.dot(x_ref[...], w_ref[...],
                  preferred_element_type=jnp.float32) + b_ref[...]
    ze1_ref[...] = ze1
    nn = _nn_from_d(_distance(ze1, emb_ref[...], en_ref[...]))
    nn1_ref[...] = nn[:, None]


def _stage_body(zeprev_ref, zqprev_ref, emb_ref, en_ref, ze_ref, nn_ref):
    ze = zeprev_ref[...] - zqprev_ref[...]
    ze_ref[...] = ze
    nn = _nn_from_d(_distance(ze, emb_ref[...], en_ref[...]))
    nn_ref[...] = nn[:, None]


@functools.cache
def _make_sc_gather():
    mesh = plsc.VectorSubcoreMesh(core_axis_name="c", subcore_axis_name="s")

    @functools.partial(
        pl.kernel,
        out_type=jax.ShapeDtypeStruct((N, CODE_DIM), jnp.float32),
        mesh=mesh,
        scratch_types=[
            pltpu.VMEM((_B_PER_W,), jnp.int32),
            pltpu.VMEM((_B_PER_W, CODE_DIM), jnp.float32),
            pltpu.SemaphoreType.DMA,
        ],
    )
    def _sc_gather(table_hbm, idx_hbm, out_hbm, idx_v, rows_v, sem):
        wid = lax.axis_index("s") * 2 + lax.axis_index("c")
        base = wid * _B_PER_W
        pltpu.sync_copy(idx_hbm.at[pl.ds(base, _B_PER_W)], idx_v)
        pltpu.async_copy(table_hbm.at[idx_v], rows_v, sem).wait()
        pltpu.sync_copy(rows_v, out_hbm.at[pl.ds(base, _B_PER_W)])

    return _sc_gather


def _dec_body(ze1_ref, zq1_ref, zq2_ref, zq3_ref, zq4_ref,
              w_ref, b_ref, out_ref):
    ze1 = ze1_ref[...]
    s = ((zq1_ref[...] + zq2_ref[...]) + zq3_ref[...]) + zq4_ref[...]
    di = ze1 + (-ze1 + s)
    out_ref[...] = jnp.dot(di, w_ref[...],
                           preferred_element_type=jnp.float32) + b_ref[...]


def _row_spec(cols):
    return pl.BlockSpec((BLK, cols), lambda i: (i, 0))


def _full_spec(rows, cols):
    return pl.BlockSpec((rows, cols), lambda i: (0, 0))


_GRID = N // BLK
_ZE = jax.ShapeDtypeStruct((N, CODE_DIM), jnp.float32)
_NN = jax.ShapeDtypeStruct((N, 1), jnp.int32)


@jax.jit
def kernel(x, enc_w, enc_b, emb1, emb2, emb3, emb4, emb5, dec_w, dec_b):
    del emb5
    enc_b2 = enc_b.reshape(1, CODE_DIM)
    dec_b2 = dec_b.reshape(1, INPUT_DIM)

    _EN = jax.ShapeDtypeStruct((1, K), jnp.float32)
    en1, en2, en3, en4 = pl.pallas_call(
        _enorm_body,
        in_specs=[pl.BlockSpec((K, CODE_DIM), lambda: (0, 0))] * 4,
        out_specs=[pl.BlockSpec((1, K), lambda: (0, 0))] * 4,
        out_shape=[_EN] * 4,
    )(emb1, emb2, emb3, emb4)

    ze1, nn1 = pl.pallas_call(
        _stage1_body,
        grid=(_GRID,),
        in_specs=[_row_spec(INPUT_DIM), _full_spec(INPUT_DIM, CODE_DIM),
                  _full_spec(1, CODE_DIM), _full_spec(K, CODE_DIM),
                  _full_spec(1, K)],
        out_specs=[_row_spec(CODE_DIM), _row_spec(1)],
        out_shape=[_ZE, _NN],
    )(x, enc_w, enc_b2, emb1, en1)
    zq1 = _make_sc_gather()(emb1, nn1.reshape(N))

    stage = pl.pallas_call(
        _stage_body,
        grid=(_GRID,),
        in_specs=[_row_spec(CODE_DIM), _row_spec(CODE_DIM),
                  _full_spec(K, CODE_DIM), _full_spec(1, K)],
        out_specs=[_row_spec(CODE_DIM), _row_spec(1)],
        out_shape=[_ZE, _NN],
    )
    ze2, nn2 = stage(ze1, zq1, emb2, en2)
    zq2 = _make_sc_gather()(emb2, nn2.reshape(N))
    ze3, nn3 = stage(ze2, zq2, emb3, en3)
    zq3 = _make_sc_gather()(emb3, nn3.reshape(N))
    ze4, nn4 = stage(ze3, zq3, emb4, en4)
    zq4 = _make_sc_gather()(emb4, nn4.reshape(N))

    x_hat = pl.pallas_call(
        _dec_body,
        grid=(_GRID,),
        in_specs=[_row_spec(CODE_DIM)] * 5
        + [_full_spec(CODE_DIM, INPUT_DIM), _full_spec(1, INPUT_DIM)],
        out_specs=_row_spec(INPUT_DIM),
        out_shape=jax.ShapeDtypeStruct((N, INPUT_DIM), jnp.float32),
    )(ze1, zq1, zq2, zq3, zq4, dec_w, dec_b2)

    r = lambda a: a.reshape(N)
    return (x_hat, ze1, ze2, ze3, ze4, zq1, zq2, zq3, zq4,
            r(nn1), r(nn2), r(nn3), r(nn4))
